# baseline (speedup 1.0000x reference)
import functools

import jax
import jax.numpy as jnp
from jax import lax
from jax.experimental import pallas as pl
from jax.experimental.pallas import tpu as pltpu
from jax.experimental.pallas import tpu_sc as plsc

VOCAB = 1000000
PVOCAB = 100000
SVOCAB = 100000
D = 50
C = 5
H = 128
T = 50
B = 16384

NC = 2
NS = 16
NW = NC * NS
N = B * C
NPW = N // NW
G = 128
NG = NPW // G


def _sc_gather(inputs_3d, w2p, w2s, emb, pre_emb, suf_emb):
    mesh = plsc.VectorSubcoreMesh(core_axis_name="c", subcore_axis_name="s")
    out_sds = jax.ShapeDtypeStruct((N, D), jnp.float32)

    @functools.partial(
        pl.kernel,
        out_type=(out_sds, out_sds, out_sds),
        mesh=mesh,
        scratch_types=[
            pltpu.VMEM((NG, G), jnp.int32),
            pltpu.VMEM((G,), jnp.int32),
            pltpu.VMEM((G,), jnp.int32),
            pltpu.VMEM((G, D), jnp.float32),
            pltpu.VMEM((G, D), jnp.float32),
            pltpu.VMEM((G, D), jnp.float32),
            pltpu.SemaphoreType.DMA,
            pltpu.SemaphoreType.DMA,
            pltpu.SemaphoreType.DMA,
        ],
        compiler_params=pltpu.CompilerParams(use_tc_tiling_on_sc=False),
    )
    def body(inputs_hbm, w2p_hbm, w2s_hbm, emb_hbm, pre_hbm, suf_hbm,
             cw_hbm, cp_hbm, cs_hbm,
             idx_all, pidx, sidx, rw, rp, rs, semw, semp, sems):
        wid = lax.axis_index("s") * NC + lax.axis_index("c")
        pltpu.sync_copy(inputs_hbm.at[wid], idx_all)

        def group(g, carry):
            idxrow = idx_all.at[g]
            cp_w = pltpu.async_copy(emb_hbm.at[idxrow], rw, semw)
            cp_p0 = pltpu.async_copy(w2p_hbm.at[idxrow], pidx, semp)
            cp_s0 = pltpu.async_copy(w2s_hbm.at[idxrow], sidx, sems)
            cp_p0.wait()
            cp_p = pltpu.async_copy(pre_hbm.at[pidx], rp, semp)
            cp_s0.wait()
            cp_s = pltpu.async_copy(suf_hbm.at[sidx], rs, sems)
            row0 = wid * NPW + g * G
            cp_w.wait()
            pltpu.sync_copy(rw, cw_hbm.at[pl.ds(row0, G)])
            cp_p.wait()
            pltpu.sync_copy(rp, cp_hbm.at[pl.ds(row0, G)])
            cp_s.wait()
            pltpu.sync_copy(rs, cs_hbm.at[pl.ds(row0, G)])
            return carry

        lax.fori_loop(0, NG, group, 0)

    return body(inputs_3d, w2p, w2s, emb, pre_emb, suf_emb)


def _mlp_body(cw_ref, cp_ref, cs_ref, w1_ref, b1_ref, w2_ref, b2_ref, out_ref):
    comp = cw_ref[...] + cp_ref[...] + cs_ref[...]
    h = jnp.tanh(
        jnp.dot(comp, w1_ref[...], preferred_element_type=jnp.float32)
        + b1_ref[...])
    o = (jnp.dot(h, w2_ref[...], preferred_element_type=jnp.float32)
         + b2_ref[...])
    m = jnp.max(o, axis=1, keepdims=True)
    x = o - m
    lse = jnp.log(jnp.sum(jnp.exp(x), axis=1, keepdims=True))
    out_ref[...] = x - lse


def _mlp(cw, cp, cs, W1, b1, W2, b2):
    BLK = 1024
    grid = (B // BLK,)
    return pl.pallas_call(
        _mlp_body,
        grid=grid,
        in_specs=[
            pl.BlockSpec((BLK, C * D), lambda i: (i, 0)),
            pl.BlockSpec((BLK, C * D), lambda i: (i, 0)),
            pl.BlockSpec((BLK, C * D), lambda i: (i, 0)),
            pl.BlockSpec((C * D, H), lambda i: (0, 0)),
            pl.BlockSpec((1, H), lambda i: (0, 0)),
            pl.BlockSpec((H, T), lambda i: (0, 0)),
            pl.BlockSpec((1, T), lambda i: (0, 0)),
        ],
        out_specs=pl.BlockSpec((BLK, T), lambda i: (i, 0)),
        out_shape=jax.ShapeDtypeStruct((B, T), jnp.float32),
    )(cw, cp, cs, W1, b1, W2, b2)


def kernel(inputs, word2prefix, word2suffix, emb, pre_emb, suf_emb,
           W1, b1, W2, b2):
    idx3 = inputs.astype(jnp.int32).reshape(NW, NG, G)
    cw, cpre, csuf = _sc_gather(idx3, word2prefix, word2suffix,
                                emb, pre_emb, suf_emb)
    cw = cw.reshape(B, C * D)
    cpre = cpre.reshape(B, C * D)
    csuf = csuf.reshape(B, C * D)
    return _mlp(cw, cpre, csuf, W1, b1.reshape(1, H), W2, b2.reshape(1, T))


# speedup vs baseline: 1.4766x; 1.4766x over previous
import functools

import jax
import jax.numpy as jnp
from jax import lax
from jax.experimental import pallas as pl
from jax.experimental.pallas import tpu as pltpu
from jax.experimental.pallas import tpu_sc as plsc

VOCAB = 1000000
PVOCAB = 100000
SVOCAB = 100000
D = 50
C = 5
H = 128
T = 50
B = 16384
DP = 128

NC = 2
NS = 16
NW = NC * NS
N = B * C
NPW = N // NW
G = 128
NG = NPW // G


def _sc_gather(iflat, w2p, w2s, embp, prep, sufp):
    mesh = plsc.VectorSubcoreMesh(core_axis_name="c", subcore_axis_name="s")
    out_sds = jax.ShapeDtypeStruct((N, DP), jnp.float32)

    @functools.partial(
        pl.kernel,
        out_type=(out_sds, out_sds, out_sds),
        mesh=mesh,
        scratch_types=[
            pltpu.VMEM((G,), jnp.int32),
            pltpu.VMEM((G,), jnp.int32),
            pltpu.VMEM((G,), jnp.int32),
            pltpu.VMEM((G, DP), jnp.float32),
            pltpu.VMEM((G, DP), jnp.float32),
            pltpu.VMEM((G, DP), jnp.float32),
            pltpu.SemaphoreType.DMA,
            pltpu.SemaphoreType.DMA,
            pltpu.SemaphoreType.DMA,
        ],
        compiler_params=pltpu.CompilerParams(use_tc_tiling_on_sc=False),
    )
    def body(iflat_hbm, w2p_hbm, w2s_hbm, emb_hbm, pre_hbm, suf_hbm,
             ew_hbm, pw_hbm, sw_hbm,
             idx, pidx, sidx, re, rp, rs, semw, semp, sems):
        wid = lax.axis_index("s") * NC + lax.axis_index("c")
        base = wid * NPW

        def group(g, carry):
            row0 = base + g * G
            pltpu.sync_copy(iflat_hbm.at[pl.ds(row0, G)], idx)
            cw = pltpu.async_copy(emb_hbm.at[idx], re, semw)
            cp0 = pltpu.async_copy(w2p_hbm.at[idx], pidx, semp)
            cs0 = pltpu.async_copy(w2s_hbm.at[idx], sidx, sems)
            cp0.wait()
            cp = pltpu.async_copy(pre_hbm.at[pidx], rp, semp)
            cs0.wait()
            cs = pltpu.async_copy(suf_hbm.at[sidx], rs, sems)
            cw.wait()
            pltpu.sync_copy(re, ew_hbm.at[pl.ds(row0, G)])
            cp.wait()
            pltpu.sync_copy(rp, pw_hbm.at[pl.ds(row0, G)])
            cs.wait()
            pltpu.sync_copy(rs, sw_hbm.at[pl.ds(row0, G)])
            return carry

        lax.fori_loop(0, NG, group, 0)

    return body(iflat, w2p, w2s, embp, prep, sufp)


def _mlp_body(ew_ref, pw_ref, sw_ref, w1_ref, b1_ref, w2_ref, b2_ref, out_ref):
    x = ew_ref[...] + pw_ref[...] + sw_ref[...]
    h = b1_ref[...]
    acc = jnp.dot(x[0], w1_ref[0], preferred_element_type=jnp.float32)
    for c in range(1, C):
        acc = acc + jnp.dot(x[c], w1_ref[c],
                            preferred_element_type=jnp.float32)
    h = jnp.tanh(acc + h)
    o = (jnp.dot(h, w2_ref[...], preferred_element_type=jnp.float32)
         + b2_ref[...])
    m = jnp.max(o, axis=1, keepdims=True)
    z = o - m
    lse = jnp.log(jnp.sum(jnp.exp(z), axis=1, keepdims=True))
    out_ref[...] = z - lse


def _mlp(ew, pw, sw, W1c, b1, W2, b2):
    BLK = 2048
    grid = (B // BLK,)
    row_spec = pl.BlockSpec((C, BLK, DP), lambda i: (0, i, 0))
    return pl.pallas_call(
        _mlp_body,
        grid=grid,
        in_specs=[
            row_spec, row_spec, row_spec,
            pl.BlockSpec((C, DP, H), lambda i: (0, 0, 0)),
            pl.BlockSpec((1, H), lambda i: (0, 0)),
            pl.BlockSpec((H, T), lambda i: (0, 0)),
            pl.BlockSpec((1, T), lambda i: (0, 0)),
        ],
        out_specs=pl.BlockSpec((BLK, T), lambda i: (i, 0)),
        out_shape=jax.ShapeDtypeStruct((B, T), jnp.float32),
    )(ew, pw, sw, W1c, b1, W2, b2)


def kernel(inputs, word2prefix, word2suffix, emb, pre_emb, suf_emb,
           W1, b1, W2, b2):
    iflat = inputs.astype(jnp.int32).T.reshape(N)
    embp = jnp.pad(emb, ((0, 0), (0, DP - D)))
    prep = jnp.pad(pre_emb, ((0, 0), (0, DP - D)))
    sufp = jnp.pad(suf_emb, ((0, 0), (0, DP - D)))
    W1c = jnp.pad(W1.reshape(C, D, H), ((0, 0), (0, DP - D), (0, 0)))

    ew, pw, sw = _sc_gather(iflat, word2prefix, word2suffix,
                            embp, prep, sufp)
    ew = ew.reshape(C, B, DP)
    pw = pw.reshape(C, B, DP)
    sw = sw.reshape(C, B, DP)
    return _mlp(ew, pw, sw, W1c, b1.reshape(1, H), W2, b2.reshape(1, T))


# speedup vs baseline: 1.4863x; 1.0066x over previous
import functools

import jax
import jax.numpy as jnp
from jax import lax
from jax.experimental import pallas as pl
from jax.experimental.pallas import tpu as pltpu
from jax.experimental.pallas import tpu_sc as plsc

VOCAB = 1000000
PVOCAB = 100000
SVOCAB = 100000
D = 50
C = 5
H = 128
T = 50
B = 16384
DP = 128

NC = 2
NS = 16
NW = NC * NS
N = B * C
NPW = N // NW
G = 128
NG = NPW // G


def _sc_gather(iflat, w2p, w2s, embp, prep, sufp):
    mesh = plsc.VectorSubcoreMesh(core_axis_name="c", subcore_axis_name="s")
    out_sds = jax.ShapeDtypeStruct((N, DP), jnp.float32)

    @functools.partial(
        pl.kernel,
        out_type=(out_sds, out_sds, out_sds),
        mesh=mesh,
        scratch_types=[
            pltpu.VMEM((NG, G), jnp.int32),
            pltpu.VMEM((NG, G), jnp.int32),
            pltpu.VMEM((NG, G), jnp.int32),
            pltpu.VMEM((2, G, DP), jnp.float32),
            pltpu.VMEM((2, G, DP), jnp.float32),
            pltpu.VMEM((2, G, DP), jnp.float32),
            pltpu.SemaphoreType.DMA,
            pltpu.SemaphoreType.DMA,
            pltpu.SemaphoreType.DMA,
            pltpu.SemaphoreType.DMA,
            pltpu.SemaphoreType.DMA,
            pltpu.SemaphoreType.DMA,
            pltpu.SemaphoreType.DMA,
            pltpu.SemaphoreType.DMA,
        ],
        compiler_params=pltpu.CompilerParams(use_tc_tiling_on_sc=False),
    )
    def body(iflat_hbm, w2p_hbm, w2s_hbm, emb_hbm, pre_hbm, suf_hbm,
             ew_hbm, pw_hbm, sw_hbm,
             idx_all, pidx_all, sidx_all, ebuf, pbuf, sbuf,
             semp, sems, sem_e, sem_pre, sem_suf, semw_e, semw_p, semw_s):
        wid = lax.axis_index("s") * NC + lax.axis_index("c")
        base = wid * NPW

        pltpu.sync_copy(iflat_hbm.at[wid], idx_all)
        cp_p = [pltpu.async_copy(w2p_hbm.at[idx_all.at[g]], pidx_all.at[g],
                                 semp) for g in range(NG)]
        cp_s = [pltpu.async_copy(w2s_hbm.at[idx_all.at[g]], sidx_all.at[g],
                                 sems) for g in range(NG)]

        gat = [None] * NG
        wrt = [None] * NG

        def fire(g):
            slot = g % 2
            if g >= 2:
                for c in wrt[g - 2]:
                    c.wait()
            cp_p[g].wait()
            cp_s[g].wait()
            gat[g] = (
                pltpu.async_copy(emb_hbm.at[idx_all.at[g]], ebuf.at[slot],
                                 sem_e),
                pltpu.async_copy(pre_hbm.at[pidx_all.at[g]], pbuf.at[slot],
                                 sem_pre),
                pltpu.async_copy(suf_hbm.at[sidx_all.at[g]], sbuf.at[slot],
                                 sem_suf),
            )

        def drain(g):
            slot = g % 2
            row0 = base + g * G
            ce, cp, cs = gat[g]
            ce.wait()
            we = pltpu.async_copy(ebuf.at[slot], ew_hbm.at[pl.ds(row0, G)],
                                  semw_e)
            cp.wait()
            wp = pltpu.async_copy(pbuf.at[slot], pw_hbm.at[pl.ds(row0, G)],
                                  semw_p)
            cs.wait()
            ws = pltpu.async_copy(sbuf.at[slot], sw_hbm.at[pl.ds(row0, G)],
                                  semw_s)
            wrt[g] = (we, wp, ws)

        fire(0)
        for g in range(1, NG):
            fire(g)
            drain(g - 1)
        drain(NG - 1)
        for g in (NG - 2, NG - 1):
            for c in wrt[g]:
                c.wait()

    iflat3 = iflat.reshape(NW, NG, G)
    return body(iflat3, w2p, w2s, embp, prep, sufp)


def _mlp_body(ew_ref, pw_ref, sw_ref, w1_ref, b1_ref, w2_ref, b2_ref, out_ref):
    x = ew_ref[...] + pw_ref[...] + sw_ref[...]
    h = b1_ref[...]
    acc = jnp.dot(x[0], w1_ref[0], preferred_element_type=jnp.float32)
    for c in range(1, C):
        acc = acc + jnp.dot(x[c], w1_ref[c],
                            preferred_element_type=jnp.float32)
    h = jnp.tanh(acc + h)
    o = (jnp.dot(h, w2_ref[...], preferred_element_type=jnp.float32)
         + b2_ref[...])
    m = jnp.max(o, axis=1, keepdims=True)
    z = o - m
    lse = jnp.log(jnp.sum(jnp.exp(z), axis=1, keepdims=True))
    out_ref[...] = z - lse


def _mlp(ew, pw, sw, W1c, b1, W2, b2):
    BLK = 2048
    grid = (B // BLK,)
    row_spec = pl.BlockSpec((C, BLK, DP), lambda i: (0, i, 0))
    return pl.pallas_call(
        _mlp_body,
        grid=grid,
        in_specs=[
            row_spec, row_spec, row_spec,
            pl.BlockSpec((C, DP, H), lambda i: (0, 0, 0)),
            pl.BlockSpec((1, H), lambda i: (0, 0)),
            pl.BlockSpec((H, T), lambda i: (0, 0)),
            pl.BlockSpec((1, T), lambda i: (0, 0)),
        ],
        out_specs=pl.BlockSpec((BLK, T), lambda i: (i, 0)),
        out_shape=jax.ShapeDtypeStruct((B, T), jnp.float32),
    )(ew, pw, sw, W1c, b1, W2, b2)


def kernel(inputs, word2prefix, word2suffix, emb, pre_emb, suf_emb,
           W1, b1, W2, b2):
    iflat = inputs.astype(jnp.int32).T.reshape(N)
    embp = jnp.pad(emb, ((0, 0), (0, DP - D)))
    prep = jnp.pad(pre_emb, ((0, 0), (0, DP - D)))
    sufp = jnp.pad(suf_emb, ((0, 0), (0, DP - D)))
    W1c = jnp.pad(W1.reshape(C, D, H), ((0, 0), (0, DP - D), (0, 0)))

    ew, pw, sw = _sc_gather(iflat, word2prefix, word2suffix,
                            embp, prep, sufp)
    ew = ew.reshape(C, B, DP)
    pw = pw.reshape(C, B, DP)
    sw = sw.reshape(C, B, DP)
    return _mlp(ew, pw, sw, W1c, b1.reshape(1, H), W2, b2.reshape(1, T))
